# Initial kernel scaffold; baseline (speedup 1.0000x reference)
import jax
import jax.numpy as jnp
from jax.experimental import pallas as pl


def kernel(x, gate_weights, noise_weights, noise):
    raise NotImplementedError("write your pallas kernel here")


# speedup vs baseline: 3.8495x; 3.8495x over previous
import jax
import jax.numpy as jnp
from jax.experimental import pallas as pl


def _router_body(x_ref, w_ref, n_ref, o_ref):
    e = n_ref.shape[1]
    logits = jnp.dot(x_ref[...], w_ref[...], preferred_element_type=jnp.float32)
    clean = logits[:, :e]
    ns = jax.nn.softplus(logits[:, e:])
    ew = clean + n_ref[...] * ns
    col = jax.lax.broadcasted_iota(jnp.int32, ew.shape, 1)
    m1 = jnp.max(ew, axis=1, keepdims=True)
    i1 = jnp.min(jnp.where(ew == m1, col, e), axis=1, keepdims=True)
    is1 = col == i1
    ew2 = jnp.where(is1, -jnp.inf, ew)
    m2 = jnp.max(ew2, axis=1, keepdims=True)
    i2 = jnp.min(jnp.where(ew2 == m2, col, e), axis=1, keepdims=True)
    e2 = jnp.exp(m2 - m1)
    inv = 1.0 / (1.0 + e2)
    o_ref[...] = jnp.where(is1, inv, jnp.where(col == i2, e2 * inv, 0.0))


def kernel(x, gate_weights, noise_weights, noise):
    n_tokens, d_model = x.shape
    n_experts = gate_weights.shape[0]
    w = jnp.concatenate([gate_weights, noise_weights], axis=0).T
    bt = 512
    return pl.pallas_call(
        _router_body,
        grid=(n_tokens // bt,),
        in_specs=[
            pl.BlockSpec((bt, d_model), lambda i: (i, 0)),
            pl.BlockSpec((d_model, 2 * n_experts), lambda i: (0, 0)),
            pl.BlockSpec((bt, n_experts), lambda i: (i, 0)),
        ],
        out_specs=pl.BlockSpec((bt, n_experts), lambda i: (i, 0)),
        out_shape=jax.ShapeDtypeStruct((n_tokens, n_experts), jnp.float32),
    )(x, w, noise)


# speedup vs baseline: 4.3164x; 1.1213x over previous
import jax
import jax.numpy as jnp
from jax.experimental import pallas as pl


def _router_body(x_ref, w_ref, n_ref, o_ref):
    e = n_ref.shape[1]
    logits = jnp.dot(x_ref[...], w_ref[...], preferred_element_type=jnp.float32)
    clean = logits[:, :e]
    ns = jax.nn.softplus(logits[:, e:])
    ew = clean + n_ref[...] * ns
    col = jax.lax.broadcasted_iota(jnp.int32, ew.shape, 1)
    m1 = jnp.max(ew, axis=1, keepdims=True)
    i1 = jnp.min(jnp.where(ew == m1, col, e), axis=1, keepdims=True)
    is1 = col == i1
    ew2 = jnp.where(is1, -jnp.inf, ew)
    m2 = jnp.max(ew2, axis=1, keepdims=True)
    i2 = jnp.min(jnp.where(ew2 == m2, col, e), axis=1, keepdims=True)
    e2 = jnp.exp(m2 - m1)
    inv = 1.0 / (1.0 + e2)
    o_ref[...] = jnp.where(is1, inv, jnp.where(col == i2, e2 * inv, 0.0))


def kernel(x, gate_weights, noise_weights, noise):
    n_tokens, d_model = x.shape
    n_experts = gate_weights.shape[0]
    w = jnp.concatenate([gate_weights, noise_weights], axis=0).T
    bt = 1024
    return pl.pallas_call(
        _router_body,
        grid=(n_tokens // bt,),
        in_specs=[
            pl.BlockSpec((bt, d_model), lambda i: (i, 0)),
            pl.BlockSpec((d_model, 2 * n_experts), lambda i: (0, 0)),
            pl.BlockSpec((bt, n_experts), lambda i: (i, 0)),
        ],
        out_specs=pl.BlockSpec((bt, n_experts), lambda i: (i, 0)),
        out_shape=jax.ShapeDtypeStruct((n_tokens, n_experts), jnp.float32),
    )(x, w, noise)
